# Initial kernel scaffold; baseline (speedup 1.0000x reference)
import jax
import jax.numpy as jnp
from jax.experimental import pallas as pl


def kernel(x, adj_c, W1, b1, W2, b2):
    raise NotImplementedError("write your pallas kernel here")


# speedup vs baseline: 12.7665x; 12.7665x over previous
import functools

import jax
import jax.numpy as jnp
from jax import lax
from jax.experimental import pallas as pl
from jax.experimental.pallas import tpu as pltpu
from jax.experimental.pallas import tpu_sc as plsc

N = 10000
E = 320000
D = 128
C = 16

NC = 2
NS = 16
NW = NC * NS
EPT = E // NW
CHUNK = 80
NCHUNK = EPT // CHUNK
NPAD = 10240
RPT = NPAD // NS
ZR = 128
ZROW = NPAD // NS

_MESH = plsc.VectorSubcoreMesh(core_axis_name="c", subcore_axis_name="s")


@functools.partial(
    pl.kernel,
    out_type=jax.ShapeDtypeStruct((NC, NPAD), jnp.float32),
    mesh=_MESH,
    scratch_types=[
        pltpu.VMEM((CHUNK,), jnp.int32),
        pltpu.VMEM((CHUNK,), jnp.float32),
        pltpu.VMEM((ZROW,), jnp.float32),
        pltpu.VMEM_SHARED((NPAD,), jnp.float32),
    ],
)
def _deg_kernel(dst_hbm, out_hbm, idx_v, ones_v, zero_v, acc_sh):
    cid = lax.axis_index("c")
    sid = lax.axis_index("s")
    wid = sid * NC + cid

    for i in range(CHUNK // 16):
        ones_v[pl.ds(i * 16, 16)] = jnp.full((16,), 1.0, jnp.float32)
    zv = jnp.zeros((16,), jnp.float32)
    for i in range(ZROW // 16):
        zero_v[pl.ds(i * 16, 16)] = zv

    pltpu.sync_copy(zero_v, acc_sh.at[pl.ds(sid * ZROW, ZROW)])
    plsc.subcore_barrier()

    def body(c, carry):
        off = wid * EPT + c * CHUNK
        pltpu.sync_copy(dst_hbm.at[pl.ds(off, CHUNK)], idx_v)
        pltpu.sync_copy(ones_v, acc_sh.at[idx_v], add=True)
        return carry

    lax.fori_loop(0, NCHUNK, body, 0)
    plsc.subcore_barrier()
    pltpu.sync_copy(acc_sh.at[pl.ds(sid * ZROW, ZROW)],
                    out_hbm.at[cid, pl.ds(sid * ZROW, ZROW)])


def _make_spmm(d, stage_table):
    scratch = [
        pltpu.VMEM((CHUNK,), jnp.int32),
        pltpu.VMEM((CHUNK,), jnp.int32),
        pltpu.VMEM((CHUNK, d), jnp.float32),
        pltpu.VMEM((ZR, d), jnp.float32),
        pltpu.VMEM_SHARED((NPAD, d), jnp.float32),
        pltpu.SemaphoreType.DMA,
    ]
    if stage_table:
        scratch.append(pltpu.VMEM_SHARED((NPAD, d), jnp.float32))

    @functools.partial(
        pl.kernel,
        out_type=jax.ShapeDtypeStruct((NC, NPAD, d), jnp.float32),
        mesh=_MESH,
        scratch_types=scratch,
    )
    def spmm(table_hbm, src_hbm, dst_hbm, out_hbm,
             sidx, didx, rows, zrows, acc, sem, *maybe_tbl):
        cid = lax.axis_index("c")
        sid = lax.axis_index("s")
        wid = sid * NC + cid

        zv = jnp.zeros((16,), jnp.float32)

        def zero_body(i, carry):
            for j in range(d // 16):
                zrows[i, pl.ds(j * 16, 16)] = zv
            return carry

        lax.fori_loop(0, ZR, zero_body, 0)
        for r in range(RPT // ZR):
            pltpu.sync_copy(zrows, acc.at[pl.ds(sid * RPT + r * ZR, ZR)])

        if stage_table:
            tbl = maybe_tbl[0]
            pltpu.sync_copy(table_hbm.at[pl.ds(sid * RPT, RPT)],
                            tbl.at[pl.ds(sid * RPT, RPT)])
            gather_src = tbl
        else:
            gather_src = table_hbm
        plsc.subcore_barrier()

        def body(c, carry):
            off = wid * EPT + c * CHUNK
            pltpu.sync_copy(src_hbm.at[pl.ds(off, CHUNK)], sidx)
            pltpu.sync_copy(dst_hbm.at[pl.ds(off, CHUNK)], didx)
            pltpu.async_copy(gather_src.at[sidx], rows, sem).wait()
            pltpu.sync_copy(rows, acc.at[didx], add=True)
            return carry

        lax.fori_loop(0, NCHUNK, body, 0)
        plsc.subcore_barrier()
        pltpu.sync_copy(acc.at[pl.ds(sid * RPT, RPT)],
                        out_hbm.at[cid, pl.ds(sid * RPT, RPT)])

    return spmm


_spmm_d = _make_spmm(D, stage_table=False)
_spmm_c = _make_spmm(C, stage_table=True)


_RB = 2000
_GRID = N // _RB


def _dinv_body(deg_ref, out_ref):
    d = deg_ref[0] + deg_ref[1]
    out_ref[...] = lax.rsqrt(jnp.maximum(d, 1.0))


def _dinv_call(degp):
    return pl.pallas_call(
        _dinv_body,
        out_shape=jax.ShapeDtypeStruct((NPAD // 128, 128), jnp.float32),
    )(degp.reshape(NC, NPAD // 128, 128))


def _l1_body(x_ref, w_ref, dinv_ref, out_ref):
    xr = x_ref[...]
    nrm = jnp.sqrt(jnp.sum(xr * xr, axis=1, keepdims=True))
    xn = xr / (nrm + 1e-12)
    m = jnp.dot(xn, w_ref[...], preferred_element_type=jnp.float32)
    out_ref[...] = m * dinv_ref[...]


def _l1_call(x, w1, dinv2d):
    return pl.pallas_call(
        _l1_body,
        grid=(_GRID,),
        in_specs=[
            pl.BlockSpec((_RB, D), lambda i: (i, 0)),
            pl.BlockSpec((D, D), lambda i: (0, 0)),
            pl.BlockSpec((_RB, 1), lambda i: (i, 0)),
        ],
        out_specs=pl.BlockSpec((_RB, D), lambda i: (i, 0)),
        out_shape=jax.ShapeDtypeStruct((N, D), jnp.float32),
    )(x, w1, dinv2d)


def _l2_body(p_ref, dinv_ref, b1_ref, w2_ref, out_ref):
    dinv = dinv_ref[...]
    h = jnp.maximum((p_ref[0] + p_ref[1]) * dinv + b1_ref[...], 0.0)
    m = jnp.dot(h, w2_ref[...], preferred_element_type=jnp.float32)
    out_ref[...] = m * dinv


_RB2 = 1280
_GRID2 = NPAD // _RB2


def _l2_call(p, dinv2d, b1, w2):
    return pl.pallas_call(
        _l2_body,
        grid=(_GRID2,),
        in_specs=[
            pl.BlockSpec((NC, _RB2, D), lambda i: (0, i, 0)),
            pl.BlockSpec((_RB2, 1), lambda i: (i, 0)),
            pl.BlockSpec((1, D), lambda i: (0, 0)),
            pl.BlockSpec((D, C), lambda i: (0, 0)),
        ],
        out_specs=pl.BlockSpec((_RB2, C), lambda i: (i, 0)),
        out_shape=jax.ShapeDtypeStruct((NPAD, C), jnp.float32),
    )(p, dinv2d, b1.reshape(1, D), w2)


def _l3_body(q_ref, dinv_ref, b2_ref, out_ref):
    logits = (q_ref[0] + q_ref[1]) * dinv_ref[...] + b2_ref[...]
    z = logits - jnp.max(logits, axis=1, keepdims=True)
    e = jnp.exp(z)
    out_ref[...] = e / jnp.sum(e, axis=1, keepdims=True)


def _l3_call(q, dinv2d, b2):
    return pl.pallas_call(
        _l3_body,
        grid=(_GRID,),
        in_specs=[
            pl.BlockSpec((NC, _RB, C), lambda i: (0, i, 0)),
            pl.BlockSpec((_RB, 1), lambda i: (i, 0)),
            pl.BlockSpec((1, C), lambda i: (0, 0)),
        ],
        out_specs=pl.BlockSpec((_RB, C), lambda i: (i, 0)),
        out_shape=jax.ShapeDtypeStruct((N, C), jnp.float32),
    )(q, dinv2d, b2.reshape(1, C))


def kernel(x, adj_c, W1, b1, W2, b2):
    src = adj_c[0]
    dst = adj_c[1]

    degp = _deg_kernel(dst)
    dinv = _dinv_call(degp)
    dinv2d = dinv.reshape(NPAD, 1)

    m1 = _l1_call(x, W1, dinv2d)
    p1 = _spmm_d(m1, src, dst)
    m2 = _l2_call(p1, dinv2d, b1, W2)
    p2 = _spmm_c(m2, src, dst)
    return _l3_call(p2, dinv2d, b2)


# speedup vs baseline: 24.7621x; 1.9396x over previous
import functools

import jax
import jax.numpy as jnp
from jax import lax
from jax.experimental import pallas as pl
from jax.experimental.pallas import tpu as pltpu
from jax.experimental.pallas import tpu_sc as plsc

N = 10000
E = 320000
D = 128
C = 16

NC = 2
NS = 16
NW = NC * NS
EPT = E // NW
CHUNK = 80
NCHUNK = EPT // CHUNK
NPAD = 10240
RPT = NPAD // NS
ZR = 128
ZROW = NPAD // NS

_MESH = plsc.VectorSubcoreMesh(core_axis_name="c", subcore_axis_name="s")


@functools.partial(
    pl.kernel,
    out_type=jax.ShapeDtypeStruct((NC, NPAD), jnp.float32),
    mesh=_MESH,
    scratch_types=[
        pltpu.VMEM((NCHUNK, CHUNK), jnp.int32),
        pltpu.VMEM((CHUNK,), jnp.float32),
        pltpu.VMEM((ZROW,), jnp.float32),
        pltpu.VMEM_SHARED((NPAD,), jnp.float32),
    ],
)
def _deg_kernel(dst_hbm, out_hbm, didx_all, ones_v, zero_v, acc_sh):
    cid = lax.axis_index("c")
    sid = lax.axis_index("s")
    wid = sid * NC + cid

    for i in range(CHUNK // 16):
        ones_v[pl.ds(i * 16, 16)] = jnp.full((16,), 1.0, jnp.float32)
    zv = jnp.zeros((16,), jnp.float32)
    for i in range(ZROW // 16):
        zero_v[pl.ds(i * 16, 16)] = zv

    pltpu.sync_copy(zero_v, acc_sh.at[pl.ds(sid * ZROW, ZROW)])
    pltpu.sync_copy(dst_hbm.at[wid], didx_all)
    plsc.subcore_barrier()

    def body(c, carry):
        pltpu.sync_copy(ones_v, acc_sh.at[didx_all.at[c]], add=True)
        return carry

    lax.fori_loop(0, NCHUNK, body, 0)
    plsc.subcore_barrier()
    pltpu.sync_copy(acc_sh.at[pl.ds(sid * ZROW, ZROW)],
                    out_hbm.at[cid, pl.ds(sid * ZROW, ZROW)])


def _make_spmm(d, stage_table):
    scratch = [
        pltpu.VMEM((CHUNK,), jnp.int32),
        pltpu.VMEM((CHUNK,), jnp.int32),
        pltpu.VMEM((CHUNK,), jnp.int32),
        pltpu.VMEM((CHUNK,), jnp.int32),
        pltpu.VMEM((CHUNK, d), jnp.float32),
        pltpu.VMEM((CHUNK, d), jnp.float32),
        pltpu.VMEM_SHARED((NPAD, d), jnp.float32),
        pltpu.SemaphoreType.DMA,
        pltpu.SemaphoreType.DMA,
        pltpu.SemaphoreType.DMA,
        pltpu.SemaphoreType.DMA,
    ]
    if stage_table:
        scratch.append(pltpu.VMEM_SHARED((NPAD, d), jnp.float32))

    @functools.partial(
        pl.kernel,
        out_type=jax.ShapeDtypeStruct((NC, NPAD, d), jnp.float32),
        mesh=_MESH,
        scratch_types=scratch,
    )
    def spmm(table_hbm, src_hbm, dst_hbm, out_hbm,
             si0, di0, si1, di1, rows0, rows1, acc,
             is0, is1, gs0, gs1, *maybe_tbl):
        cid = lax.axis_index("c")
        sid = lax.axis_index("s")
        wid = sid * NC + cid

        zv = jnp.zeros((16,), jnp.float32)

        def zero_body(i, carry):
            for j in range(d // 16):
                rows0[i, pl.ds(j * 16, 16)] = zv
            return carry

        lax.fori_loop(0, CHUNK, zero_body, 0)
        for r in range(RPT // CHUNK):
            pltpu.sync_copy(rows0, acc.at[pl.ds(sid * RPT + r * CHUNK, CHUNK)])

        if stage_table:
            tbl = maybe_tbl[0]
            pltpu.sync_copy(table_hbm.at[pl.ds(sid * RPT, RPT)],
                            tbl.at[pl.ds(sid * RPT, RPT)])
            gather_src = tbl
        else:
            gather_src = table_hbm
        plsc.subcore_barrier()

        sets = ((si0, di0, is0), (si1, di1, is1))
        gathers = ((rows0, gs0), (rows1, gs1))

        def idx_load(c, k):
            si, di, sem = sets[k]
            pltpu.async_copy(src_hbm.at[wid, c], si, sem)
            pltpu.async_copy(dst_hbm.at[wid, c], di, sem)

        def idx_wait(c, k):
            si, di, sem = sets[k]
            pltpu.make_async_copy(src_hbm.at[wid, c], si, sem).wait()
            pltpu.make_async_copy(dst_hbm.at[wid, c], di, sem).wait()

        def gather(k, j):
            rows, sem = gathers[j]
            pltpu.async_copy(gather_src.at[sets[k][0]], rows, sem)

        def gather_wait(k, j):
            rows, sem = gathers[j]
            pltpu.make_async_copy(gather_src.at[sets[k][0]], rows, sem).wait()

        def scatter(k, j):
            pltpu.sync_copy(gathers[j][0], acc.at[sets[k][1]], add=True)

        idx_load(0, 0)
        idx_wait(0, 0)
        gather(0, 0)
        idx_load(1, 1)

        def half(c0, cnext, a, b):
            idx_wait(c0 + 1, b)
            gather_wait(a, a)
            gather(b, b)
            scatter(a, a)
            idx_load(cnext, a)

        def body(g, carry):
            c0 = 2 * g
            half(c0, c0 + 2, 0, 1)
            half(c0 + 1, jnp.minimum(c0 + 3, NCHUNK - 1), 1, 0)
            return carry

        lax.fori_loop(0, (NCHUNK - 1) // 2, body, 0)
        gather_wait(0, 0)
        scatter(0, 0)

        plsc.subcore_barrier()
        pltpu.sync_copy(acc.at[pl.ds(sid * RPT, RPT)],
                        out_hbm.at[cid, pl.ds(sid * RPT, RPT)])

    return spmm


_spmm_d = _make_spmm(D, stage_table=False)
_spmm_c = _make_spmm(C, stage_table=True)


_RB = 2000
_GRID = N // _RB


def _dinv_of(deg_ref):
    return lax.rsqrt(jnp.maximum(deg_ref[0] + deg_ref[1], 1.0))


def _l1_body(x_ref, w_ref, out_ref):
    xr = x_ref[...]
    nrm = jnp.sqrt(jnp.sum(xr * xr, axis=1, keepdims=True))
    xn = xr / (nrm + 1e-12)
    out_ref[...] = jnp.dot(xn, w_ref[...], preferred_element_type=jnp.float32)


def _l1_call(x, w1):
    return pl.pallas_call(
        _l1_body,
        grid=(_GRID,),
        in_specs=[
            pl.BlockSpec((_RB, D), lambda i: (i, 0)),
            pl.BlockSpec((D, D), lambda i: (0, 0)),
        ],
        out_specs=pl.BlockSpec((_RB, D), lambda i: (i, 0)),
        out_shape=jax.ShapeDtypeStruct((N, D), jnp.float32),
    )(x, w1)


def _scale_body(m_ref, deg_ref, out_ref):
    out_ref[...] = m_ref[...] * _dinv_of(deg_ref)


def _scale_call(m1, degp3):
    return pl.pallas_call(
        _scale_body,
        grid=(_GRID,),
        in_specs=[
            pl.BlockSpec((_RB, D), lambda i: (i, 0)),
            pl.BlockSpec((NC, _RB, 1), lambda i: (0, i, 0)),
        ],
        out_specs=pl.BlockSpec((_RB, D), lambda i: (i, 0)),
        out_shape=jax.ShapeDtypeStruct((N, D), jnp.float32),
    )(m1, degp3)


def _l2_body(p_ref, deg_ref, b1_ref, w2_ref, out_ref):
    dinv = _dinv_of(deg_ref)
    h = jnp.maximum((p_ref[0] + p_ref[1]) * dinv + b1_ref[...], 0.0)
    m = jnp.dot(h, w2_ref[...], preferred_element_type=jnp.float32)
    out_ref[...] = m * dinv


_RB2 = 1280
_GRID2 = NPAD // _RB2


def _l2_call(p, degp3, b1, w2):
    return pl.pallas_call(
        _l2_body,
        grid=(_GRID2,),
        in_specs=[
            pl.BlockSpec((NC, _RB2, D), lambda i: (0, i, 0)),
            pl.BlockSpec((NC, _RB2, 1), lambda i: (0, i, 0)),
            pl.BlockSpec((1, D), lambda i: (0, 0)),
            pl.BlockSpec((D, C), lambda i: (0, 0)),
        ],
        out_specs=pl.BlockSpec((_RB2, C), lambda i: (i, 0)),
        out_shape=jax.ShapeDtypeStruct((NPAD, C), jnp.float32),
    )(p, degp3, b1.reshape(1, D), w2)


def _l3_body(q_ref, deg_ref, b2_ref, out_ref):
    logits = (q_ref[0] + q_ref[1]) * _dinv_of(deg_ref) + b2_ref[...]
    z = logits - jnp.max(logits, axis=1, keepdims=True)
    e = jnp.exp(z)
    out_ref[...] = e / jnp.sum(e, axis=1, keepdims=True)


def _l3_call(q, degp3, b2):
    return pl.pallas_call(
        _l3_body,
        grid=(_GRID,),
        in_specs=[
            pl.BlockSpec((NC, _RB, C), lambda i: (0, i, 0)),
            pl.BlockSpec((NC, _RB, 1), lambda i: (0, i, 0)),
            pl.BlockSpec((1, C), lambda i: (0, 0)),
        ],
        out_specs=pl.BlockSpec((_RB, C), lambda i: (i, 0)),
        out_shape=jax.ShapeDtypeStruct((N, C), jnp.float32),
    )(q, degp3, b2.reshape(1, C))


def kernel(x, adj_c, W1, b1, W2, b2):
    src = adj_c[0].reshape(NW, NCHUNK, CHUNK)
    dst = adj_c[1].reshape(NW, NCHUNK, CHUNK)

    degp = _deg_kernel(dst)
    m1 = _l1_call(x, W1)
    degp3 = degp.reshape(NC, NPAD, 1)
    t1 = _scale_call(m1, degp3)
    p1 = _spmm_d(t1, src, dst)
    m2 = _l2_call(p1, degp3, b1, W2)
    p2 = _spmm_c(m2, src, dst)
    return _l3_call(p2, degp3, b2)
